# Initial kernel scaffold; baseline (speedup 1.0000x reference)
import jax
import jax.numpy as jnp
from jax.experimental import pallas as pl


def kernel(keys, points, feats, n_idxs, neighbor_rel, neighbor_valid, W1, b1, W2, b2):
    raise NotImplementedError("write your pallas kernel here")


# speedup vs baseline: 10.6447x; 10.6447x over previous
import functools

import jax
import jax.numpy as jnp
from jax import lax
from jax.experimental import pallas as pl
from jax.experimental.pallas import tpu as pltpu
from jax.experimental.pallas import tpu_sc as plsc

B, N, K, C, H, O = 2, 10000, 16, 128, 128, 128
BN = B * N
NW = 32
NODES_PER_TILE = BN // NW
NODE_CHUNK = 5
CHUNKS = NODES_PER_TILE // NODE_CHUNK
ROW_BLK = 1000
LANES = 16
JV = H // LANES


def _proj_body(f_ref, w1a_ref, w1b_ref, b1_ref, p_ref, q_ref):
    f = f_ref[...]
    p_ref[...] = jnp.dot(f, w1a_ref[...], preferred_element_type=jnp.float32) + b1_ref[...]
    q_ref[...] = jnp.dot(f, w1b_ref[...], preferred_element_type=jnp.float32)


def _project(feats2, w1a, w1b, b1row):
    return pl.pallas_call(
        _proj_body,
        grid=(BN // ROW_BLK,),
        in_specs=[
            pl.BlockSpec((ROW_BLK, C), lambda i: (i, 0)),
            pl.BlockSpec((C, H), lambda i: (0, 0)),
            pl.BlockSpec((C, H), lambda i: (0, 0)),
            pl.BlockSpec((1, H), lambda i: (0, 0)),
        ],
        out_specs=[
            pl.BlockSpec((ROW_BLK, H), lambda i: (i, 0)),
            pl.BlockSpec((ROW_BLK, H), lambda i: (i, 0)),
        ],
        out_shape=[
            jax.ShapeDtypeStruct((BN, H), jnp.float32),
            jax.ShapeDtypeStruct((BN, H), jnp.float32),
        ],
    )(feats2, w1a, w1b, b1row)


def _sc_body(q_hbm, p_hbm, gidx_hbm, valid_hbm, hsum_hbm,
             idx_v, rows_v, p_v, valid_v, out_v, sem):
    wid = lax.axis_index("s") * 2 + lax.axis_index("c")
    base = wid * NODES_PER_TILE

    def chunk_body(ci, carry):
        nb = base + ci * NODE_CHUNK
        pltpu.sync_copy(gidx_hbm.at[pl.ds(nb * K, NODE_CHUNK * K)], idx_v)
        pltpu.sync_copy(valid_hbm.at[pl.ds(nb * K, NODE_CHUNK * K)], valid_v)
        pltpu.sync_copy(p_hbm.at[pl.ds(nb * H, NODE_CHUNK * H)], p_v)
        pltpu.async_copy(q_hbm.at[idx_v], rows_v, sem).wait()

        def node_body(i, c2):
            vvec = valid_v[pl.ds(i * K, K)]
            vks = [jnp.full((LANES,), vvec[kk], jnp.float32)
                   for kk in range(K)]
            for j in range(JV):
                pj = p_v[pl.ds(i * H + j * LANES, LANES)]
                acc = jnp.zeros((LANES,), jnp.float32)
                for kk in range(K):
                    qk = rows_v[i * K + kk, pl.ds(j * LANES, LANES)]
                    acc = acc + vks[kk] * jnp.maximum(pj + qk, 0.0)
                out_v[pl.ds(i * H + j * LANES, LANES)] = acc
            return c2

        lax.fori_loop(0, NODE_CHUNK, node_body, 0)
        pltpu.sync_copy(out_v, hsum_hbm.at[pl.ds(nb * H, NODE_CHUNK * H)])
        return carry

    lax.fori_loop(0, CHUNKS, chunk_body, 0)


def _sc_gather_reduce(q2, p2, gidx, valid1):
    mesh = plsc.VectorSubcoreMesh(core_axis_name="c", subcore_axis_name="s")
    fn = pl.kernel(
        _sc_body,
        out_type=jax.ShapeDtypeStruct((BN * H,), jnp.float32),
        mesh=mesh,
        scratch_types=[
            pltpu.VMEM((NODE_CHUNK * K,), jnp.int32),
            pltpu.VMEM((NODE_CHUNK * K, H), jnp.float32),
            pltpu.VMEM((NODE_CHUNK * H,), jnp.float32),
            pltpu.VMEM((NODE_CHUNK * K,), jnp.float32),
            pltpu.VMEM((NODE_CHUNK * H,), jnp.float32),
            pltpu.SemaphoreType.DMA,
        ],
    )
    return fn(q2, p2.reshape(BN * H), gidx, valid1).reshape(BN, H)


def _out_body(h_ref, v_ref, w2_ref, b2rep_ref, o_ref):
    o_ref[...] = (jnp.dot(h_ref[...], w2_ref[...], preferred_element_type=jnp.float32)
                  + jnp.dot(v_ref[...], b2rep_ref[...], preferred_element_type=jnp.float32))


def _finish(hsum, valid2, w2, b2rep):
    return pl.pallas_call(
        _out_body,
        grid=(BN // ROW_BLK,),
        in_specs=[
            pl.BlockSpec((ROW_BLK, H), lambda i: (i, 0)),
            pl.BlockSpec((ROW_BLK, K), lambda i: (i, 0)),
            pl.BlockSpec((H, O), lambda i: (0, 0)),
            pl.BlockSpec((K, O), lambda i: (0, 0)),
        ],
        out_specs=pl.BlockSpec((ROW_BLK, O), lambda i: (i, 0)),
        out_shape=jax.ShapeDtypeStruct((BN, O), jnp.float32),
    )(hsum, valid2, w2, b2rep)


def kernel(keys, points, feats, n_idxs, neighbor_rel, neighbor_valid, W1, b1, W2, b2):
    feats2 = feats.reshape(BN, C)
    w1a = W1[:C]
    w1b = W1[C:]
    b1row = b1.reshape(1, H)
    p2, q2 = _project(feats2, w1a, w1b, b1row)

    gidx = (n_idxs.astype(jnp.int32)
            + (jnp.arange(B, dtype=jnp.int32) * N)[:, None, None]).reshape(BN * K)
    valid1 = neighbor_valid.reshape(BN * K)
    hsum = _sc_gather_reduce(q2, p2, gidx, valid1)

    b2rep = jnp.broadcast_to(b2[None, :], (K, O))
    out = _finish(hsum, neighbor_valid.reshape(BN, K), W2, b2rep)
    return out.reshape(B, N, O)
